# baseline (speedup 1.0000x reference)
import functools

import jax
import jax.numpy as jnp
import numpy as np
from jax import lax
from jax.experimental import pallas as pl
from jax.experimental.pallas import tpu as pltpu
from jax.experimental.pallas import tpu_sc as plsc

_SEQ = 512
_ANS = 8
_B = 4
_C = 384
_P = 4068
_PPAD = 4080
_NCHUNK = _PPAD // 16
_NFULL = 4032 // 16


def _tail_tables():
    r = np.arange(_SEQ)
    d = r[None, :] - r[:, None]
    m = (d >= 0) & (d < _ANS)
    si, ei = np.nonzero(m)
    sit = np.zeros((48,), np.int32)
    eit = np.zeros((48,), np.int32)
    sit[: _P - 4032] = si[4032:]
    eit[: _P - 4032] = ei[4032:]
    return jnp.asarray(sit), jnp.asarray(eit)


def _proj_body(emb_ref, tt_ref, am_ref, w_ref, b_ref, out_ref):
    emb = emb_ref[...]
    w = w_ref[...]
    mask = (tt_ref[...] * am_ref[...]).astype(jnp.float32)
    pen = -1e7 * (1.0 - mask)
    S = jnp.sum(emb[:, :, :_C] * w[None, :, :], axis=-1) + b_ref[0, 0] + pen
    E = jnp.sum(emb[:, :, _C:] * w[None, :, :], axis=-1) + pen
    out_ref[...] = jnp.concatenate([S, E], axis=0)


def _sc_body(sem_hbm, sit_hbm, eit_hbm, out_hbm,
             srow, erow, stail, etail, ebuf):
    s = lax.axis_index("s")

    if True:
        row = s
        pltpu.sync_copy(sem_hbm.at[row % 4], srow)
        pltpu.sync_copy(sem_hbm.at[4 + row // 4], erow)
        pltpu.sync_copy(sit_hbm, stail)
        pltpu.sync_copy(eit_hbm, etail)
        lane = lax.iota(jnp.int32, 16)
        dconst = lane & 7
        sbase = lax.shift_right_logical(lane, 3)
        two = jnp.full((16,), 2, jnp.int32)

        def bodyA(k, carry):
            acc, sidx = carry
            sv = plsc.load_gather(srow, [sidx])
            ev = plsc.load_gather(erow, [sidx + dconst])
            ex = jnp.exp(sv + ev)
            ebuf[pl.ds(k * 16, 16)] = ex
            return acc + ex, sidx + two

        acc, _ = lax.fori_loop(
            0, _NFULL, bodyA,
            (jnp.zeros((16,), jnp.float32), sbase), unroll=12)
        for kk in range(3):
            sidx = stail[pl.ds(kk * 16, 16)]
            eidx = etail[pl.ds(kk * 16, 16)]
            sv = plsc.load_gather(srow, [sidx])
            ev = plsc.load_gather(erow, [eidx])
            ex = jnp.exp(sv + ev)
            base = 4032 + kk * 16
            ex = jnp.where(base + lane < _P, ex, 0.0)
            ebuf[pl.ds(base, 16)] = ex
            acc = acc + ex
        ssum = jnp.sum(acc)
        rinv = (jnp.full((16,), 1.0, jnp.float32)
                / jnp.broadcast_to(ssum, (16,)))

        def bodyB(k, carry):
            ebuf[pl.ds(k * 16, 16)] = ebuf[pl.ds(k * 16, 16)] * rinv
            return carry

        lax.fori_loop(0, _NCHUNK, bodyB, 0, unroll=5)
        pltpu.sync_copy(ebuf, out_hbm.at[row])


_sc_call = functools.partial(
    pl.kernel,
    mesh=plsc.VectorSubcoreMesh(core_axis_name="c", subcore_axis_name="s",
                                num_cores=1),
    compiler_params=pltpu.CompilerParams(needs_layout_passes=False),
    out_type=jax.ShapeDtypeStruct((16, _PPAD), jnp.float32),
    scratch_types=[
        pltpu.VMEM((_SEQ,), jnp.float32),
        pltpu.VMEM((_SEQ,), jnp.float32),
        pltpu.VMEM((48,), jnp.int32),
        pltpu.VMEM((48,), jnp.int32),
        pltpu.VMEM((_PPAD,), jnp.float32),
    ],
)(_sc_body)


def kernel(embeddings, token_type_ids, attention_mask, W, b):
    tt = token_type_ids.reshape(1, _SEQ)
    am = attention_mask.reshape(1, _SEQ)
    wr = W.reshape(1, _C)
    br = b.reshape(1, 1)
    sem = pl.pallas_call(
        _proj_body,
        out_shape=jax.ShapeDtypeStruct((8, _SEQ), jnp.float32),
    )(embeddings, tt, am, wr, br)
    sit, eit = _tail_tables()
    out = _sc_call(sem, sit, eit)
    return out[:, :_P].reshape(_B, _B, _P)


# speedup vs baseline: 1.0578x; 1.0578x over previous
import functools

import jax
import jax.numpy as jnp
import numpy as np
from jax import lax
from jax.experimental import pallas as pl
from jax.experimental.pallas import tpu as pltpu
from jax.experimental.pallas import tpu_sc as plsc

_SEQ = 512
_ANS = 8
_B = 4
_C = 384
_P = 4068
_PPAD = 4080
_NCHUNK = _PPAD // 16
_NFULL = 4032 // 16


def _tail_row():
    r = np.arange(_SEQ)
    d = r[None, :] - r[:, None]
    m = (d >= 0) & (d < _ANS)
    si, ei = np.nonzero(m)
    row = np.zeros((_SEQ,), np.int32)
    row[: _P - 4032] = si[4032:]
    row[48 : 48 + _P - 4032] = ei[4032:]
    return jnp.asarray(row.view(np.float32)).reshape(1, _SEQ)


def _proj_body(emb_ref, tt_ref, am_ref, w_ref, b_ref, tbl_ref, out_ref):
    emb = emb_ref[...]
    w = w_ref[...]
    mask = (tt_ref[...] * am_ref[...]).astype(jnp.float32)
    pen = -1e7 * (1.0 - mask)
    S = jnp.sum(emb[:, :, :_C] * w[None, :, :], axis=-1) + b_ref[0, 0] + pen
    E = jnp.sum(emb[:, :, _C:] * w[None, :, :], axis=-1) + pen
    out_ref[...] = jnp.concatenate(
        [S, E, tbl_ref[...], jnp.zeros((7, _SEQ), jnp.float32)], axis=0)


def _sc_body(sem_hbm, out_hbm, tab, ebuf):
    s = lax.axis_index("s")
    row = s
    pltpu.sync_copy(sem_hbm, tab)
    lane = lax.iota(jnp.int32, 16)
    dconst = lane & 7
    sbase = lax.shift_right_logical(lane, 3)
    two = jnp.full((16,), 2, jnp.int32)
    srow_id = jnp.broadcast_to(row % 4, (16,))
    erow_id = jnp.broadcast_to(4 + row // 4, (16,))

    def bodyA(k, carry):
        acc, sidx = carry
        sv = plsc.load_gather(tab, [srow_id, sidx])
        ev = plsc.load_gather(tab, [erow_id, sidx + dconst])
        ex = jnp.exp(sv + ev)
        ebuf[pl.ds(k * 16, 16)] = ex
        return acc + ex, sidx + two

    acc, _ = lax.fori_loop(
        0, _NFULL, bodyA,
        (jnp.zeros((16,), jnp.float32), sbase), unroll=12)
    for kk in range(3):
        sidx = plsc.bitcast(tab[8, pl.ds(kk * 16, 16)], jnp.int32)
        eidx = plsc.bitcast(tab[8, pl.ds(48 + kk * 16, 16)], jnp.int32)
        sv = plsc.load_gather(tab, [srow_id, sidx])
        ev = plsc.load_gather(tab, [erow_id, eidx])
        ex = jnp.exp(sv + ev)
        base = 4032 + kk * 16
        ex = jnp.where(base + lane < _P, ex, 0.0)
        ebuf[pl.ds(base, 16)] = ex
        acc = acc + ex
    ssum = jnp.sum(acc)
    rinv = (jnp.full((16,), 1.0, jnp.float32)
            / jnp.broadcast_to(ssum, (16,)))

    def bodyB(k, carry):
        ebuf[pl.ds(k * 16, 16)] = ebuf[pl.ds(k * 16, 16)] * rinv
        return carry

    lax.fori_loop(0, _NCHUNK, bodyB, 0, unroll=5)
    pltpu.sync_copy(ebuf, out_hbm.at[row])


_sc_call = functools.partial(
    pl.kernel,
    mesh=plsc.VectorSubcoreMesh(core_axis_name="c", subcore_axis_name="s",
                                num_cores=1),
    compiler_params=pltpu.CompilerParams(needs_layout_passes=False),
    out_type=jax.ShapeDtypeStruct((16, _PPAD), jnp.float32),
    scratch_types=[
        pltpu.VMEM((16, _SEQ), jnp.float32),
        pltpu.VMEM((_PPAD,), jnp.float32),
    ],
)(_sc_body)


def kernel(embeddings, token_type_ids, attention_mask, W, b):
    tt = token_type_ids.reshape(1, _SEQ)
    am = attention_mask.reshape(1, _SEQ)
    wr = W.reshape(1, _C)
    br = b.reshape(1, 1)
    sem = pl.pallas_call(
        _proj_body,
        out_shape=jax.ShapeDtypeStruct((16, _SEQ), jnp.float32),
    )(embeddings, tt, am, wr, br, _tail_row())
    out = _sc_call(sem)
    return out[:, :_P].reshape(_B, _B, _P)


# speedup vs baseline: 1.1726x; 1.1086x over previous
import functools

import jax
import jax.numpy as jnp
import numpy as np
from jax import lax
from jax.experimental import pallas as pl
from jax.experimental.pallas import tpu as pltpu
from jax.experimental.pallas import tpu_sc as plsc

_SEQ = 512
_ANS = 8
_B = 4
_C = 384
_P = 4068
_PPAD = 4080
_NCHUNK = _PPAD // 16
_NBLK = 31
_TAILP = _NBLK * 128


def _tail_row():
    r = np.arange(_SEQ)
    d = r[None, :] - r[:, None]
    m = (d >= 0) & (d < _ANS)
    si, ei = np.nonzero(m)
    row = np.zeros((_SEQ,), np.int32)
    row[: _P - _TAILP] = si[_TAILP:]
    row[112 : 112 + _P - _TAILP] = ei[_TAILP:]
    return jnp.asarray(row.view(np.float32)).reshape(1, _SEQ)


def _proj_body(emb_ref, tt_ref, am_ref, w_ref, b_ref, tbl_ref, out_ref):
    emb = emb_ref[...]
    w = w_ref[...]
    mask = (tt_ref[...] * am_ref[...]).astype(jnp.float32)
    pen = -1e7 * (1.0 - mask)
    S = jnp.sum(emb[:, :, :_C] * w[None, :, :], axis=-1) + b_ref[0, 0] + pen
    E = jnp.sum(emb[:, :, _C:] * w[None, :, :], axis=-1) + pen
    out_ref[...] = jnp.concatenate(
        [S, E, tbl_ref[...], jnp.zeros((7, _SEQ), jnp.float32)], axis=0)


def _vgather(vec, idx):
    return lax.gather(
        vec, idx[:, None],
        dimension_numbers=lax.GatherDimensionNumbers(
            offset_dims=(), collapsed_slice_dims=(0,), start_index_map=(0,)),
        slice_sizes=(1,),
        mode=lax.GatherScatterMode.PROMISE_IN_BOUNDS)


def _sc_body(sem_hbm, out_hbm, tab, ebuf):
    s = lax.axis_index("s")
    row = s
    pltpu.sync_copy(sem_hbm, tab)
    lane = lax.iota(jnp.int32, 16)
    srow_id = jnp.broadcast_to(row % 4, (16,))
    erow_id = jnp.broadcast_to(4 + row // 4, (16,))
    sbase = lax.shift_right_logical(lane, 3)
    eexp = sbase + (lane & 7)

    jrow = row % 4
    irow = 4 + row // 4

    def bodyA(blk, acc):
        t0 = blk * 16
        s16 = tab[jrow, pl.ds(t0, 16)]
        e16a = tab[irow, pl.ds(t0, 16)]
        e16b = tab[irow, pl.ds(t0 + 8, 16)]
        for c in range(8):
            sv = _vgather(s16, sbase + 2 * c)
            ev = _vgather(e16a if c < 4 else e16b,
                          eexp + (2 * c - (8 if c >= 4 else 0)))
            ex = jnp.exp(sv + ev)
            ebuf[pl.ds(blk * 128 + c * 16, 16)] = ex
            acc = acc + ex
        return acc

    acc = lax.fori_loop(0, _NBLK, bodyA, jnp.zeros((16,), jnp.float32))
    for kk in range(7):
        sidx = plsc.bitcast(tab[8, pl.ds(kk * 16, 16)], jnp.int32)
        eidx = plsc.bitcast(tab[8, pl.ds(112 + kk * 16, 16)], jnp.int32)
        sv = plsc.load_gather(tab, [srow_id, sidx])
        ev = plsc.load_gather(tab, [erow_id, eidx])
        ex = jnp.exp(sv + ev)
        base = _TAILP + kk * 16
        ex = jnp.where(base + lane < _P, ex, 0.0)
        ebuf[pl.ds(base, 16)] = ex
        acc = acc + ex
    ssum = jnp.sum(acc)
    rinv = (jnp.full((16,), 1.0, jnp.float32)
            / jnp.broadcast_to(ssum, (16,)))

    def bodyB(k, carry):
        ebuf[pl.ds(k * 16, 16)] = ebuf[pl.ds(k * 16, 16)] * rinv
        return carry

    lax.fori_loop(0, _NCHUNK, bodyB, 0, unroll=5)
    pltpu.sync_copy(ebuf, out_hbm.at[row])


_sc_call = functools.partial(
    pl.kernel,
    mesh=plsc.VectorSubcoreMesh(core_axis_name="c", subcore_axis_name="s",
                                num_cores=1),
    compiler_params=pltpu.CompilerParams(needs_layout_passes=False),
    out_type=jax.ShapeDtypeStruct((16, _PPAD), jnp.float32),
    scratch_types=[
        pltpu.VMEM((16, _SEQ), jnp.float32),
        pltpu.VMEM((_PPAD,), jnp.float32),
    ],
)(_sc_body)


def kernel(embeddings, token_type_ids, attention_mask, W, b):
    tt = token_type_ids.reshape(1, _SEQ)
    am = attention_mask.reshape(1, _SEQ)
    wr = W.reshape(1, _C)
    br = b.reshape(1, 1)
    sem = pl.pallas_call(
        _proj_body,
        out_shape=jax.ShapeDtypeStruct((16, _SEQ), jnp.float32),
    )(embeddings, tt, am, wr, br, _tail_row())
    out = _sc_call(sem)
    return out[:, :_P].reshape(_B, _B, _P)
